# Initial kernel scaffold; baseline (speedup 1.0000x reference)
import jax
import jax.numpy as jnp
from jax.experimental import pallas as pl


def kernel(index, value, W_self, b_self, W_row, b_row, W_col, b_col, W_glob, b_glob, W, b):
    raise NotImplementedError("write your pallas kernel here")


# speedup vs baseline: 9.2676x; 9.2676x over previous
import functools

import jax
import jax.numpy as jnp
from jax import lax
from jax.experimental import pallas as pl
from jax.experimental.pallas import tpu as pltpu
from jax.experimental.pallas import tpu_sc as plsc

N_SEG = 10000
N_PAD = 10240
NC = 2
NS = 16
LANES = 16
CHUNK = 256
KIDX = CHUNK // 128


def _zero_rows(ref, nrows, ncols):
    zv = jnp.zeros((LANES,), jnp.float32)

    def body(i, _):
        for j in range(ncols // LANES):
            ref[i, pl.ds(j * LANES, LANES)] = zv
        return 0

    lax.fori_loop(0, nrows, body, 0)


SCHUNK = 128


def _make_sc_segment_sum(E, D):
    nchunks_total = E // SCHUNK
    cpt = -(-nchunks_total // NS)
    cpt += (-cpt) % 16
    igrp = cpt // 4
    rows_per_tile = N_PAD // NS
    mesh = plsc.VectorSubcoreMesh(core_axis_name="c", subcore_axis_name="s")

    @functools.partial(
        pl.kernel,
        out_type=(
            jax.ShapeDtypeStruct((NC, N_PAD, D), jnp.float32),
            jax.ShapeDtypeStruct((NC, NS, 8, 128), jnp.float32),
        ),
        mesh=mesh,
        scratch_types=[
            pltpu.VMEM_SHARED((N_PAD, D), jnp.float32),
            pltpu.VMEM_SHARED((N_PAD,), jnp.float32),
            pltpu.VMEM((SCHUNK, D), jnp.float32),
            pltpu.VMEM((igrp, 128), jnp.int32),
            pltpu.VMEM((128,), jnp.float32),
            pltpu.VMEM((rows_per_tile,), jnp.float32),
            pltpu.VMEM((8, 128), jnp.float32),
        ],
    )
    def seg_sum(value_hbm, idxrc_hbm, s_hbm, cnt_hbm,
                s_sh, cnt_sh, vbuf, ibig, ones1, c1, cbuf):
        cid = lax.axis_index("c")
        sid = lax.axis_index("s")
        zv = jnp.zeros((LANES,), jnp.float32)
        ov = jnp.ones((LANES,), jnp.float32)

        _zero_rows(vbuf, SCHUNK, D)

        def z_body(i, _):
            c1[pl.ds(i * LANES, LANES)] = zv
            return 0

        lax.fori_loop(0, rows_per_tile // LANES, z_body, 0)
        for j in range(128 // LANES):
            ones1[pl.ds(j * LANES, LANES)] = ov
        base = sid * rows_per_tile
        for z in range(rows_per_tile // SCHUNK):
            pltpu.sync_copy(vbuf, s_sh.at[pl.ds(base + z * SCHUNK, SCHUNK)])
        pltpu.sync_copy(c1, cnt_sh.at[pl.ds(base, rows_per_tile)])
        plsc.subcore_barrier()

        nchunks = lax.max(0, lax.min(cpt, nchunks_total - sid * cpt))
        for h in range(4):
            pltpu.sync_copy(
                idxrc_hbm.at[cid].at[pl.ds(sid * cpt + h * igrp, igrp)], ibig)
            ngrp = lax.max(0, lax.min(igrp, nchunks - h * igrp))

            def chunk_body(k, _):
                g = sid * cpt + h * igrp + k
                pltpu.sync_copy(value_hbm.at[pl.ds(g * SCHUNK, SCHUNK)], vbuf)
                pltpu.sync_copy(vbuf, s_sh.at[ibig.at[k]], add=True)
                pltpu.sync_copy(ones1, cnt_sh.at[ibig.at[k]], add=True)
                return 0

            lax.fori_loop(0, ngrp, chunk_body, 0)
        plsc.subcore_barrier()

        pltpu.sync_copy(s_sh.at[pl.ds(base, rows_per_tile)],
                        s_hbm.at[cid].at[pl.ds(base, rows_per_tile)])
        pltpu.sync_copy(cnt_sh.at[pl.ds(base, rows_per_tile)], c1)

        def pack_body(i, _):
            for j in range(128 // LANES):
                cbuf[i, pl.ds(j * LANES, LANES)] = c1[pl.ds(i * 128 + j * LANES,
                                                            LANES)]
            return 0

        lax.fori_loop(0, rows_per_tile // 128, pack_body, 0)

        def zz_body(i, _):
            for j in range(128 // LANES):
                cbuf[i, pl.ds(j * LANES, LANES)] = zv
            return 0

        lax.fori_loop(rows_per_tile // 128, 8, zz_body, 0)
        pltpu.sync_copy(cbuf, cnt_hbm.at[cid, sid])

    return seg_sum


def _make_sc_gather(E, D):
    nchunks_total = E // CHUNK
    nw = NC * NS
    cpw = -(-nchunks_total // nw)
    cpw += (-cpw) % 4
    mesh = plsc.VectorSubcoreMesh(core_axis_name="c", subcore_axis_name="s")

    @functools.partial(
        pl.kernel,
        out_type=jax.ShapeDtypeStruct((E, D), jnp.float32),
        mesh=mesh,
        scratch_types=[
            pltpu.VMEM((cpw * KIDX, 128), jnp.int32),
            pltpu.VMEM((cpw * KIDX, 128), jnp.int32),
            pltpu.VMEM((CHUNK, D), jnp.float32),
            pltpu.VMEM((CHUNK, D), jnp.float32),
            pltpu.SemaphoreType.DMA,
        ],
    )
    def gather(p_hbm, idxrc_hbm, g_hbm, ibr, ibc, bufr, bufc, sem):
        cid = lax.axis_index("c")
        sid = lax.axis_index("s")
        wid = sid * NC + cid

        pltpu.sync_copy(idxrc_hbm.at[0].at[pl.ds(wid * cpw * KIDX, cpw * KIDX)],
                        ibr)
        pltpu.sync_copy(idxrc_hbm.at[1].at[pl.ds(wid * cpw * KIDX, cpw * KIDX)],
                        ibc)
        nchunks = lax.max(0, lax.min(cpw, nchunks_total - wid * cpw))

        def chunk_body(k, _):
            g = wid * cpw + k
            cps = []
            for j in range(KIDX):
                cps.append(pltpu.async_copy(
                    p_hbm.at[0].at[ibr.at[k * KIDX + j]],
                    bufr.at[pl.ds(j * 128, 128)], sem))
                cps.append(pltpu.async_copy(
                    p_hbm.at[1].at[ibc.at[k * KIDX + j]],
                    bufc.at[pl.ds(j * 128, 128)], sem))
            for cp in cps:
                cp.wait()

            def add_body(i, _):
                for j in range(D // LANES):
                    sl = pl.ds(j * LANES, LANES)
                    bufr[i, sl] = bufr[i, sl] + bufc[i, sl]
                return 0

            lax.fori_loop(0, CHUNK, add_body, 0)
            pltpu.sync_copy(bufr, g_hbm.at[pl.ds(g * CHUNK, CHUNK)])
            return 0

        lax.fori_loop(0, nchunks, chunk_body, 0)

    return gather


def _tc_node_body(s_ref, cntr_ref, cntc_ref, wself_ref, wrow_ref, wcol_ref,
                  wglob_ref, w_ref, bself_ref, brow_ref, bcol_ref, bglob_ref,
                  b_ref, e_scalar_ref, p_ref, aself_ref, cvec_ref):
    d = wself_ref.shape[0]
    w1 = w_ref[:, 0:d]
    w2 = w_ref[:, d:2 * d]
    w3 = w_ref[:, 2 * d:3 * d]
    w4 = w_ref[:, 3 * d:4 * d]
    dn = (((1,), (1,)), ((), ()))

    aself_ref[...] = jnp.dot(w1, wself_ref[...], preferred_element_type=jnp.float32)

    s_row = s_ref[0]
    s_col = s_ref[1]
    m_row = s_row / (cntr_ref[...] + 1e-9)
    m_col = s_col / (cntc_ref[...] + 1e-9)
    b2 = jnp.dot(w2, wrow_ref[...], preferred_element_type=jnp.float32)
    b3 = jnp.dot(w3, wcol_ref[...], preferred_element_type=jnp.float32)
    p_ref[0] = lax.dot_general(m_row, b2, dn, preferred_element_type=jnp.float32)
    p_ref[1] = lax.dot_general(m_col, b3, dn, preferred_element_type=jnp.float32)

    g = jnp.sum(s_row, axis=0, keepdims=True) / e_scalar_ref[0, 0]
    gl = lax.dot_general(g, wglob_ref[...], dn,
                         preferred_element_type=jnp.float32) + bglob_ref[...]
    cvec_ref[...] = (
        b_ref[...]
        + lax.dot_general(bself_ref[...], w1, dn, preferred_element_type=jnp.float32)
        + lax.dot_general(brow_ref[...], w2, dn, preferred_element_type=jnp.float32)
        + lax.dot_general(bcol_ref[...], w3, dn, preferred_element_type=jnp.float32)
        + lax.dot_general(gl, w4, dn, preferred_element_type=jnp.float32)
    )


def _tc_edge_body(v_ref, g_ref, aself_ref, cvec_ref, o_ref):
    t = lax.dot_general(v_ref[...], aself_ref[...], (((1,), (1,)), ((), ())),
                        preferred_element_type=jnp.float32)
    t = t + g_ref[...] + cvec_ref[...]
    o_ref[...] = jnp.maximum(t, 0.01 * t)


def kernel(index, value, W_self, b_self, W_row, b_row, W_col, b_col,
           W_glob, b_glob, W, b):
    B, E, D = value.shape
    v2 = value.reshape(E, D)

    cpt = -(-(E // SCHUNK) // NS)
    cpt += (-cpt) % 16
    rows_pad = cpt * NS
    cpw = -(-(E // CHUNK) // (NC * NS))
    cpw += (-cpw) % 4
    rows_pad = max(rows_pad, cpw * KIDX * NC * NS)

    idx2 = index[0].T.reshape(2, E // 128, 128)
    pad = rows_pad - E // 128
    idxrc = jnp.pad(idx2, ((0, 0), (0, pad), (0, 0)))

    s_tab, cnt_tab = _make_sc_segment_sum(E, D)(v2, idxrc)
    npt = N_PAD // NS
    cnt_n = cnt_tab.reshape(2, NS, 1024)[:, :, :npt].reshape(2, N_PAD)

    node_call = pl.pallas_call(
        _tc_node_body,
        out_shape=(
            jax.ShapeDtypeStruct((2, N_PAD, D), jnp.float32),
            jax.ShapeDtypeStruct((D, D), jnp.float32),
            jax.ShapeDtypeStruct((1, D), jnp.float32),
        ),
    )
    p_tab, a_self, cvec = node_call(
        s_tab, cnt_n[0].reshape(N_PAD, 1), cnt_n[1].reshape(N_PAD, 1),
        W_self, W_row, W_col, W_glob, W,
        b_self.reshape(1, D), b_row.reshape(1, D), b_col.reshape(1, D),
        b_glob.reshape(1, D), b.reshape(1, D),
        jnp.full((1, 1), float(E), jnp.float32),
    )

    g_sum = _make_sc_gather(E, D)(p_tab, idxrc)

    BLK = 1280
    edge_call = pl.pallas_call(
        _tc_edge_body,
        grid=(E // BLK,),
        in_specs=[
            pl.BlockSpec((BLK, D), lambda i: (i, 0)),
            pl.BlockSpec((BLK, D), lambda i: (i, 0)),
            pl.BlockSpec((D, D), lambda i: (0, 0)),
            pl.BlockSpec((1, D), lambda i: (0, 0)),
        ],
        out_specs=pl.BlockSpec((BLK, D), lambda i: (i, 0)),
        out_shape=jax.ShapeDtypeStruct((E, D), jnp.float32),
    )
    out = edge_call(v2, g_sum, a_self, cvec)
    return (index, out.reshape(B, E, D))


# speedup vs baseline: 11.9625x; 1.2908x over previous
import functools

import jax
import jax.numpy as jnp
from jax import lax
from jax.experimental import pallas as pl
from jax.experimental.pallas import tpu as pltpu
from jax.experimental.pallas import tpu_sc as plsc

N_SEG = 10000
N_PAD = 10240
NC = 2
NS = 16
LANES = 16
CHUNK = 256
KIDX = CHUNK // 128


def _zero_rows(ref, nrows, ncols):
    zv = jnp.zeros((LANES,), jnp.float32)

    def body(i, _):
        for j in range(ncols // LANES):
            ref[i, pl.ds(j * LANES, LANES)] = zv
        return 0

    lax.fori_loop(0, nrows, body, 0)


SCHUNK = 128


def _make_sc_segment_sum(E, D):
    nchunks_total = E // SCHUNK
    cpt = -(-nchunks_total // NS)
    cpt += (-cpt) % 16
    igrp = cpt // 4
    rows_per_tile = N_PAD // NS
    mesh = plsc.VectorSubcoreMesh(core_axis_name="c", subcore_axis_name="s")

    @functools.partial(
        pl.kernel,
        out_type=(
            jax.ShapeDtypeStruct((NC, N_PAD, D), jnp.float32),
            jax.ShapeDtypeStruct((NC, NS, 8, 128), jnp.float32),
        ),
        mesh=mesh,
        scratch_types=[
            pltpu.VMEM_SHARED((N_PAD, D), jnp.float32),
            pltpu.VMEM_SHARED((N_PAD,), jnp.float32),
            pltpu.VMEM((2, SCHUNK, D), jnp.float32),
            pltpu.VMEM((igrp, 128), jnp.int32),
            pltpu.VMEM((128,), jnp.float32),
            pltpu.VMEM((rows_per_tile,), jnp.float32),
            pltpu.VMEM((8, 128), jnp.float32),
            pltpu.SemaphoreType.DMA,
            pltpu.SemaphoreType.DMA,
            pltpu.SemaphoreType.DMA,
        ],
    )
    def seg_sum(value_hbm, idxrc_hbm, s_hbm, cnt_hbm,
                s_sh, cnt_sh, vbuf, ibig, ones1, c1, cbuf,
                sem_l, sem_s, sem_c):
        cid = lax.axis_index("c")
        sid = lax.axis_index("s")
        zv = jnp.zeros((LANES,), jnp.float32)
        ov = jnp.ones((LANES,), jnp.float32)

        _zero_rows(vbuf.at[0], SCHUNK, D)

        def z_body(i, _):
            c1[pl.ds(i * LANES, LANES)] = zv
            return 0

        lax.fori_loop(0, rows_per_tile // LANES, z_body, 0)
        for j in range(128 // LANES):
            ones1[pl.ds(j * LANES, LANES)] = ov
        base = sid * rows_per_tile
        for z in range(rows_per_tile // SCHUNK):
            pltpu.sync_copy(vbuf.at[0], s_sh.at[pl.ds(base + z * SCHUNK, SCHUNK)])
        pltpu.sync_copy(c1, cnt_sh.at[pl.ds(base, rows_per_tile)])
        plsc.subcore_barrier()

        nchunks = lax.max(0, lax.min(cpt, nchunks_total - sid * cpt))
        for h in range(4):
            pltpu.sync_copy(
                idxrc_hbm.at[cid].at[pl.ds(sid * cpt + h * igrp, igrp)], ibig)
            ngrp = lax.max(0, lax.min(igrp, nchunks - h * igrp))
            gbase = sid * cpt + h * igrp

            def _load(k, b):
                pltpu.async_copy(
                    value_hbm.at[pl.ds((gbase + k) * SCHUNK, SCHUNK)],
                    vbuf.at[b], sem_l)

            @pl.when(ngrp > 0)
            def _():
                _load(0, 0)

            def pair_body(i, _):
                for b in range(2):
                    k = 2 * i + b

                    @pl.when(k > 0)
                    def _():
                        pltpu.make_async_copy(
                            vbuf.at[1 - b], s_sh.at[ibig.at[k - 1]],
                            sem_s).wait()
                        pltpu.make_async_copy(
                            ones1, cnt_sh.at[ibig.at[k - 1]], sem_c).wait()

                    pltpu.make_async_copy(
                        value_hbm.at[pl.ds((gbase + k) * SCHUNK, SCHUNK)],
                        vbuf.at[b], sem_l).wait()

                    @pl.when(k + 1 < ngrp)
                    def _():
                        _load(k + 1, 1 - b)

                    pltpu.async_copy(vbuf.at[b], s_sh.at[ibig.at[k]], sem_s,
                                     add=True)
                    pltpu.async_copy(ones1, cnt_sh.at[ibig.at[k]], sem_c,
                                     add=True)
                return 0

            lax.fori_loop(0, ngrp // 2, pair_body, 0)

            @pl.when(ngrp > 0)
            def _():
                pltpu.make_async_copy(
                    vbuf.at[1], s_sh.at[ibig.at[ngrp - 1]], sem_s).wait()
                pltpu.make_async_copy(
                    ones1, cnt_sh.at[ibig.at[ngrp - 1]], sem_c).wait()
        plsc.subcore_barrier()

        pltpu.sync_copy(s_sh.at[pl.ds(base, rows_per_tile)],
                        s_hbm.at[cid].at[pl.ds(base, rows_per_tile)])
        pltpu.sync_copy(cnt_sh.at[pl.ds(base, rows_per_tile)], c1)

        def pack_body(i, _):
            for j in range(128 // LANES):
                cbuf[i, pl.ds(j * LANES, LANES)] = c1[pl.ds(i * 128 + j * LANES,
                                                            LANES)]
            return 0

        lax.fori_loop(0, rows_per_tile // 128, pack_body, 0)

        def zz_body(i, _):
            for j in range(128 // LANES):
                cbuf[i, pl.ds(j * LANES, LANES)] = zv
            return 0

        lax.fori_loop(rows_per_tile // 128, 8, zz_body, 0)
        pltpu.sync_copy(cbuf, cnt_hbm.at[cid, sid])

    return seg_sum


def _make_sc_gather(E, D):
    nchunks_total = E // SCHUNK
    nw = NC * NS
    cpw = -(-nchunks_total // nw)
    cpw += (-cpw) % 8
    mesh = plsc.VectorSubcoreMesh(core_axis_name="c", subcore_axis_name="s")

    @functools.partial(
        pl.kernel,
        out_type=jax.ShapeDtypeStruct((E, D), jnp.float32),
        mesh=mesh,
        scratch_types=[
            pltpu.VMEM((cpw, 128), jnp.int32),
            pltpu.VMEM((cpw, 128), jnp.int32),
            pltpu.VMEM((2, SCHUNK, D), jnp.float32),
            pltpu.VMEM((2, SCHUNK, D), jnp.float32),
            pltpu.SemaphoreType.DMA,
            pltpu.SemaphoreType.DMA,
            pltpu.SemaphoreType.DMA,
        ],
    )
    def gather(p_hbm, idxrc_hbm, g_hbm, ibr, ibc, bufr, bufc,
               sem_g, sem_w, sem_x):
        cid = lax.axis_index("c")
        sid = lax.axis_index("s")
        wid = sid * NC + cid

        pltpu.sync_copy(idxrc_hbm.at[0].at[pl.ds(wid * cpw, cpw)], ibr)
        pltpu.sync_copy(idxrc_hbm.at[1].at[pl.ds(wid * cpw, cpw)], ibc)
        nchunks = lax.max(0, lax.min(cpw, nchunks_total - wid * cpw))

        def _gather(k, b):
            pltpu.async_copy(p_hbm.at[0].at[ibr.at[k]], bufr.at[b], sem_g)
            pltpu.async_copy(p_hbm.at[1].at[ibc.at[k]], bufc.at[b], sem_x)

        @pl.when(nchunks > 0)
        def _():
            _gather(0, 0)

        def pair_body(i, _):
            for b in range(2):
                k = 2 * i + b
                pltpu.make_async_copy(p_hbm.at[0].at[ibr.at[k]], bufr.at[b],
                                      sem_g).wait()
                pltpu.make_async_copy(p_hbm.at[1].at[ibc.at[k]], bufc.at[b],
                                      sem_x).wait()

                @pl.when(k > 0)
                def _():
                    pltpu.make_async_copy(
                        bufr.at[1 - b],
                        g_hbm.at[pl.ds((wid * cpw + k - 1) * SCHUNK, SCHUNK)],
                        sem_w).wait()

                @pl.when(k + 1 < nchunks)
                def _():
                    _gather(k + 1, 1 - b)

                def add_body(r, _):
                    for j in range(D // LANES):
                        sl = pl.ds(j * LANES, LANES)
                        bufr[b, r, sl] = bufr[b, r, sl] + bufc[b, r, sl]
                    return 0

                lax.fori_loop(0, SCHUNK, add_body, 0)
                pltpu.async_copy(
                    bufr.at[b],
                    g_hbm.at[pl.ds((wid * cpw + k) * SCHUNK, SCHUNK)], sem_w)
            return 0

        lax.fori_loop(0, nchunks // 2, pair_body, 0)

        @pl.when(nchunks > 0)
        def _():
            pltpu.make_async_copy(
                bufr.at[1],
                g_hbm.at[pl.ds((wid * cpw + nchunks - 1) * SCHUNK, SCHUNK)],
                sem_w).wait()

    return gather


def _tc_node_body(s_ref, cntr_ref, cntc_ref, wself_ref, wrow_ref, wcol_ref,
                  wglob_ref, w_ref, bself_ref, brow_ref, bcol_ref, bglob_ref,
                  b_ref, e_scalar_ref, p_ref, aself_ref, cvec_ref):
    d = wself_ref.shape[0]
    w1 = w_ref[:, 0:d]
    w2 = w_ref[:, d:2 * d]
    w3 = w_ref[:, 2 * d:3 * d]
    w4 = w_ref[:, 3 * d:4 * d]
    dn = (((1,), (1,)), ((), ()))

    aself_ref[...] = jnp.dot(w1, wself_ref[...], preferred_element_type=jnp.float32)

    s_row = s_ref[0]
    s_col = s_ref[1]
    m_row = s_row / (cntr_ref[...] + 1e-9)
    m_col = s_col / (cntc_ref[...] + 1e-9)
    b2 = jnp.dot(w2, wrow_ref[...], preferred_element_type=jnp.float32)
    b3 = jnp.dot(w3, wcol_ref[...], preferred_element_type=jnp.float32)
    p_ref[0] = lax.dot_general(m_row, b2, dn, preferred_element_type=jnp.float32)
    p_ref[1] = lax.dot_general(m_col, b3, dn, preferred_element_type=jnp.float32)

    g = jnp.sum(s_row, axis=0, keepdims=True) / e_scalar_ref[0, 0]
    gl = lax.dot_general(g, wglob_ref[...], dn,
                         preferred_element_type=jnp.float32) + bglob_ref[...]
    cvec_ref[...] = (
        b_ref[...]
        + lax.dot_general(bself_ref[...], w1, dn, preferred_element_type=jnp.float32)
        + lax.dot_general(brow_ref[...], w2, dn, preferred_element_type=jnp.float32)
        + lax.dot_general(bcol_ref[...], w3, dn, preferred_element_type=jnp.float32)
        + lax.dot_general(gl, w4, dn, preferred_element_type=jnp.float32)
    )


def _tc_edge_body(v_ref, g_ref, aself_ref, cvec_ref, o_ref):
    t = lax.dot_general(v_ref[...], aself_ref[...], (((1,), (1,)), ((), ())),
                        preferred_element_type=jnp.float32)
    t = t + g_ref[...] + cvec_ref[...]
    o_ref[...] = jnp.maximum(t, 0.01 * t)


def kernel(index, value, W_self, b_self, W_row, b_row, W_col, b_col,
           W_glob, b_glob, W, b):
    B, E, D = value.shape
    v2 = value.reshape(E, D)

    cpt = -(-(E // SCHUNK) // NS)
    cpt += (-cpt) % 16
    rows_pad = cpt * NS
    cpw = -(-(E // SCHUNK) // (NC * NS))
    cpw += (-cpw) % 8
    rows_pad = max(rows_pad, cpw * NC * NS)

    idx2 = index[0].T.reshape(2, E // 128, 128)
    pad = rows_pad - E // 128
    idxrc = jnp.pad(idx2, ((0, 0), (0, pad), (0, 0)))

    s_tab, cnt_tab = _make_sc_segment_sum(E, D)(v2, idxrc)
    npt = N_PAD // NS
    cnt_n = cnt_tab.reshape(2, NS, 1024)[:, :, :npt].reshape(2, N_PAD)

    node_call = pl.pallas_call(
        _tc_node_body,
        out_shape=(
            jax.ShapeDtypeStruct((2, N_PAD, D), jnp.float32),
            jax.ShapeDtypeStruct((D, D), jnp.float32),
            jax.ShapeDtypeStruct((1, D), jnp.float32),
        ),
    )
    p_tab, a_self, cvec = node_call(
        s_tab, cnt_n[0].reshape(N_PAD, 1), cnt_n[1].reshape(N_PAD, 1),
        W_self, W_row, W_col, W_glob, W,
        b_self.reshape(1, D), b_row.reshape(1, D), b_col.reshape(1, D),
        b_glob.reshape(1, D), b.reshape(1, D),
        jnp.full((1, 1), float(E), jnp.float32),
    )

    g_sum = _make_sc_gather(E, D)(p_tab, idxrc)

    BLK = 1280
    edge_call = pl.pallas_call(
        _tc_edge_body,
        grid=(E // BLK,),
        in_specs=[
            pl.BlockSpec((BLK, D), lambda i: (i, 0)),
            pl.BlockSpec((BLK, D), lambda i: (i, 0)),
            pl.BlockSpec((D, D), lambda i: (0, 0)),
            pl.BlockSpec((1, D), lambda i: (0, 0)),
        ],
        out_specs=pl.BlockSpec((BLK, D), lambda i: (i, 0)),
        out_shape=jax.ShapeDtypeStruct((E, D), jnp.float32),
    )
    out = edge_call(v2, g_sum, a_self, cvec)
    return (index, out.reshape(B, E, D))


# speedup vs baseline: 15.4915x; 1.2950x over previous
import functools

import jax
import jax.numpy as jnp
from jax import lax
from jax.experimental import pallas as pl
from jax.experimental.pallas import tpu as pltpu
from jax.experimental.pallas import tpu_sc as plsc

N_SEG = 10000
N_PAD = 10240
NC = 2
NS = 16
LANES = 16
CHUNK = 256
KIDX = CHUNK // 128


def _zero_rows(ref, nrows, ncols):
    zv = jnp.zeros((LANES,), jnp.float32)

    def body(i, _):
        for j in range(ncols // LANES):
            ref[i, pl.ds(j * LANES, LANES)] = zv
        return 0

    lax.fori_loop(0, nrows, body, 0)


SCHUNK = 128


def _make_sc_segment_sum(E, D):
    nchunks_total = E // SCHUNK
    cpt = -(-nchunks_total // NS)
    cpt += (-cpt) % 16
    igrp = cpt // 4
    rows_per_tile = N_PAD // NS
    mesh = plsc.VectorSubcoreMesh(core_axis_name="c", subcore_axis_name="s")

    @functools.partial(
        pl.kernel,
        out_type=(
            jax.ShapeDtypeStruct((NC, N_PAD, D), jnp.float32),
            jax.ShapeDtypeStruct((NC, NS, 8, 128), jnp.float32),
        ),
        mesh=mesh,
        scratch_types=[
            pltpu.VMEM_SHARED((N_PAD, D), jnp.float32),
            pltpu.VMEM_SHARED((N_PAD,), jnp.float32),
            pltpu.VMEM((2, SCHUNK, D), jnp.float32),
            pltpu.VMEM((igrp, 128), jnp.int32),
            pltpu.VMEM((128,), jnp.float32),
            pltpu.VMEM((rows_per_tile,), jnp.float32),
            pltpu.VMEM((8, 128), jnp.float32),
            pltpu.SemaphoreType.DMA,
            pltpu.SemaphoreType.DMA,
            pltpu.SemaphoreType.DMA,
        ],
    )
    def seg_sum(value_hbm, idxrc_hbm, s_hbm, cnt_hbm,
                s_sh, cnt_sh, vbuf, ibig, ones1, c1, cbuf,
                sem_l, sem_s, sem_c):
        cid = lax.axis_index("c")
        sid = lax.axis_index("s")
        zv = jnp.zeros((LANES,), jnp.float32)
        ov = jnp.ones((LANES,), jnp.float32)

        _zero_rows(vbuf.at[0], SCHUNK, D)

        def z_body(i, _):
            c1[pl.ds(i * LANES, LANES)] = zv
            return 0

        lax.fori_loop(0, rows_per_tile // LANES, z_body, 0)
        for j in range(128 // LANES):
            ones1[pl.ds(j * LANES, LANES)] = ov
        base = sid * rows_per_tile
        for z in range(rows_per_tile // SCHUNK):
            pltpu.sync_copy(vbuf.at[0], s_sh.at[pl.ds(base + z * SCHUNK, SCHUNK)])
        pltpu.sync_copy(c1, cnt_sh.at[pl.ds(base, rows_per_tile)])
        plsc.subcore_barrier()

        nchunks = lax.max(0, lax.min(cpt, nchunks_total - sid * cpt))
        for h in range(4):
            pltpu.sync_copy(
                idxrc_hbm.at[cid].at[pl.ds(sid * cpt + h * igrp, igrp)], ibig)
            ngrp = lax.max(0, lax.min(igrp, nchunks - h * igrp))
            gbase = sid * cpt + h * igrp

            def _load(k, b):
                pltpu.async_copy(
                    value_hbm.at[pl.ds((gbase + k) * SCHUNK, SCHUNK)],
                    vbuf.at[b], sem_l)

            @pl.when(ngrp > 0)
            def _():
                _load(0, 0)

            def pair_body(i, _):
                for b in range(2):
                    k = 2 * i + b

                    @pl.when(k > 0)
                    def _():
                        pltpu.make_async_copy(
                            vbuf.at[1 - b], s_sh.at[ibig.at[k - 1]],
                            sem_s).wait()
                        pltpu.make_async_copy(
                            ones1, cnt_sh.at[ibig.at[k - 1]], sem_c).wait()

                    pltpu.make_async_copy(
                        value_hbm.at[pl.ds((gbase + k) * SCHUNK, SCHUNK)],
                        vbuf.at[b], sem_l).wait()

                    @pl.when(k + 1 < ngrp)
                    def _():
                        _load(k + 1, 1 - b)

                    pltpu.async_copy(vbuf.at[b], s_sh.at[ibig.at[k]], sem_s,
                                     add=True)
                    pltpu.async_copy(ones1, cnt_sh.at[ibig.at[k]], sem_c,
                                     add=True)
                return 0

            lax.fori_loop(0, ngrp // 2, pair_body, 0)

            @pl.when(ngrp > 0)
            def _():
                pltpu.make_async_copy(
                    vbuf.at[1], s_sh.at[ibig.at[ngrp - 1]], sem_s).wait()
                pltpu.make_async_copy(
                    ones1, cnt_sh.at[ibig.at[ngrp - 1]], sem_c).wait()
        plsc.subcore_barrier()

        pltpu.sync_copy(s_sh.at[pl.ds(base, rows_per_tile)],
                        s_hbm.at[cid].at[pl.ds(base, rows_per_tile)])
        pltpu.sync_copy(cnt_sh.at[pl.ds(base, rows_per_tile)], c1)

        def pack_body(i, _):
            for j in range(128 // LANES):
                cbuf[i, pl.ds(j * LANES, LANES)] = c1[pl.ds(i * 128 + j * LANES,
                                                            LANES)]
            return 0

        lax.fori_loop(0, rows_per_tile // 128, pack_body, 0)

        def zz_body(i, _):
            for j in range(128 // LANES):
                cbuf[i, pl.ds(j * LANES, LANES)] = zv
            return 0

        lax.fori_loop(rows_per_tile // 128, 8, zz_body, 0)
        pltpu.sync_copy(cbuf, cnt_hbm.at[cid, sid])

    return seg_sum


def _make_sc_combine(E, D):
    nchunks_total = E // SCHUNK
    nw = NC * NS
    cpw = -(-nchunks_total // nw)
    cpw += (-cpw) % 8
    mesh = plsc.VectorSubcoreMesh(core_axis_name="c", subcore_axis_name="s")

    @functools.partial(
        pl.kernel,
        out_type=jax.ShapeDtypeStruct((E, D), jnp.float32),
        mesh=mesh,
        scratch_types=[
            pltpu.VMEM((cpw, 128), jnp.int32),
            pltpu.VMEM((cpw, 128), jnp.int32),
            pltpu.VMEM((2, SCHUNK, D), jnp.float32),
            pltpu.VMEM((2, SCHUNK, D), jnp.float32),
            pltpu.VMEM((2, SCHUNK, D), jnp.float32),
            pltpu.SemaphoreType.DMA,
            pltpu.SemaphoreType.DMA,
            pltpu.SemaphoreType.DMA,
            pltpu.SemaphoreType.DMA,
        ],
    )
    def combine(p_hbm, idxrc_hbm, y_hbm, o_hbm, ibr, ibc, bufr, bufc, bufy,
                sem_g, sem_w, sem_x, sem_y):
        cid = lax.axis_index("c")
        sid = lax.axis_index("s")
        wid = sid * NC + cid

        pltpu.sync_copy(idxrc_hbm.at[0].at[pl.ds(wid * cpw, cpw)], ibr)
        pltpu.sync_copy(idxrc_hbm.at[1].at[pl.ds(wid * cpw, cpw)], ibc)
        nchunks = lax.max(0, lax.min(cpw, nchunks_total - wid * cpw))

        def _fetch(k, b):
            pltpu.async_copy(p_hbm.at[0].at[ibr.at[k]], bufr.at[b], sem_g)
            pltpu.async_copy(p_hbm.at[1].at[ibc.at[k]], bufc.at[b], sem_x)
            pltpu.async_copy(
                y_hbm.at[pl.ds((wid * cpw + k) * SCHUNK, SCHUNK)],
                bufy.at[b], sem_y)

        @pl.when(nchunks > 0)
        def _():
            _fetch(0, 0)

        def pair_body(i, _):
            for b in range(2):
                k = 2 * i + b
                pltpu.make_async_copy(p_hbm.at[0].at[ibr.at[k]], bufr.at[b],
                                      sem_g).wait()
                pltpu.make_async_copy(p_hbm.at[1].at[ibc.at[k]], bufc.at[b],
                                      sem_x).wait()
                pltpu.make_async_copy(
                    y_hbm.at[pl.ds((wid * cpw + k) * SCHUNK, SCHUNK)],
                    bufy.at[b], sem_y).wait()

                @pl.when(k > 0)
                def _():
                    pltpu.make_async_copy(
                        bufr.at[1 - b],
                        o_hbm.at[pl.ds((wid * cpw + k - 1) * SCHUNK, SCHUNK)],
                        sem_w).wait()

                @pl.when(k + 1 < nchunks)
                def _():
                    _fetch(k + 1, 1 - b)

                def add_body(r, _):
                    for j in range(D // LANES):
                        sl = pl.ds(j * LANES, LANES)
                        t = bufy[b, r, sl] + bufr[b, r, sl] + bufc[b, r, sl]
                        bufr[b, r, sl] = jnp.maximum(t, 0.01 * t)
                    return 0

                lax.fori_loop(0, SCHUNK, add_body, 0)
                pltpu.async_copy(
                    bufr.at[b],
                    o_hbm.at[pl.ds((wid * cpw + k) * SCHUNK, SCHUNK)], sem_w)
            return 0

        lax.fori_loop(0, nchunks // 2, pair_body, 0)

        @pl.when(nchunks > 0)
        def _():
            pltpu.make_async_copy(
                bufr.at[1],
                o_hbm.at[pl.ds((wid * cpw + nchunks - 1) * SCHUNK, SCHUNK)],
                sem_w).wait()

    return combine


def _make_sc_gather(E, D):
    nchunks_total = E // SCHUNK
    nw = NC * NS
    cpw = -(-nchunks_total // nw)
    cpw += (-cpw) % 8
    mesh = plsc.VectorSubcoreMesh(core_axis_name="c", subcore_axis_name="s")

    @functools.partial(
        pl.kernel,
        out_type=jax.ShapeDtypeStruct((E, D), jnp.float32),
        mesh=mesh,
        scratch_types=[
            pltpu.VMEM((cpw, 128), jnp.int32),
            pltpu.VMEM((cpw, 128), jnp.int32),
            pltpu.VMEM((2, SCHUNK, D), jnp.float32),
            pltpu.VMEM((2, SCHUNK, D), jnp.float32),
            pltpu.SemaphoreType.DMA,
            pltpu.SemaphoreType.DMA,
            pltpu.SemaphoreType.DMA,
        ],
    )
    def gather(p_hbm, idxrc_hbm, g_hbm, ibr, ibc, bufr, bufc,
               sem_g, sem_w, sem_x):
        cid = lax.axis_index("c")
        sid = lax.axis_index("s")
        wid = sid * NC + cid

        pltpu.sync_copy(idxrc_hbm.at[0].at[pl.ds(wid * cpw, cpw)], ibr)
        pltpu.sync_copy(idxrc_hbm.at[1].at[pl.ds(wid * cpw, cpw)], ibc)
        nchunks = lax.max(0, lax.min(cpw, nchunks_total - wid * cpw))

        def _gather(k, b):
            pltpu.async_copy(p_hbm.at[0].at[ibr.at[k]], bufr.at[b], sem_g)
            pltpu.async_copy(p_hbm.at[1].at[ibc.at[k]], bufc.at[b], sem_x)

        @pl.when(nchunks > 0)
        def _():
            _gather(0, 0)

        def pair_body(i, _):
            for b in range(2):
                k = 2 * i + b
                pltpu.make_async_copy(p_hbm.at[0].at[ibr.at[k]], bufr.at[b],
                                      sem_g).wait()
                pltpu.make_async_copy(p_hbm.at[1].at[ibc.at[k]], bufc.at[b],
                                      sem_x).wait()

                @pl.when(k > 0)
                def _():
                    pltpu.make_async_copy(
                        bufr.at[1 - b],
                        g_hbm.at[pl.ds((wid * cpw + k - 1) * SCHUNK, SCHUNK)],
                        sem_w).wait()

                @pl.when(k + 1 < nchunks)
                def _():
                    _gather(k + 1, 1 - b)

                def add_body(r, _):
                    for j in range(D // LANES):
                        sl = pl.ds(j * LANES, LANES)
                        bufr[b, r, sl] = bufr[b, r, sl] + bufc[b, r, sl]
                    return 0

                lax.fori_loop(0, SCHUNK, add_body, 0)
                pltpu.async_copy(
                    bufr.at[b],
                    g_hbm.at[pl.ds((wid * cpw + k) * SCHUNK, SCHUNK)], sem_w)
            return 0

        lax.fori_loop(0, nchunks // 2, pair_body, 0)

        @pl.when(nchunks > 0)
        def _():
            pltpu.make_async_copy(
                bufr.at[1],
                g_hbm.at[pl.ds((wid * cpw + nchunks - 1) * SCHUNK, SCHUNK)],
                sem_w).wait()

    return gather


def _tc_aself_body(wself_ref, w_ref, aself_ref):
    d = wself_ref.shape[0]
    aself_ref[...] = jnp.dot(w_ref[:, 0:d], wself_ref[...],
                             preferred_element_type=jnp.float32)


def _tc_node_body(s_ref, cntr_ref, cntc_ref, wrow_ref, wcol_ref,
                  wglob_ref, w_ref, bself_ref, brow_ref, bcol_ref, bglob_ref,
                  b_ref, e_scalar_ref, p_ref):
    d = wrow_ref.shape[0]
    w1 = w_ref[:, 0:d]
    w2 = w_ref[:, d:2 * d]
    w3 = w_ref[:, 2 * d:3 * d]
    w4 = w_ref[:, 3 * d:4 * d]
    dn = (((1,), (1,)), ((), ()))

    s_row = s_ref[0]
    s_col = s_ref[1]
    m_row = s_row / (cntr_ref[...] + 1e-9)
    m_col = s_col / (cntc_ref[...] + 1e-9)
    b2 = jnp.dot(w2, wrow_ref[...], preferred_element_type=jnp.float32)
    b3 = jnp.dot(w3, wcol_ref[...], preferred_element_type=jnp.float32)

    g = jnp.sum(s_row, axis=0, keepdims=True) / e_scalar_ref[0, 0]
    gl = lax.dot_general(g, wglob_ref[...], dn,
                         preferred_element_type=jnp.float32) + bglob_ref[...]
    cvec = (
        b_ref[...]
        + lax.dot_general(bself_ref[...], w1, dn, preferred_element_type=jnp.float32)
        + lax.dot_general(brow_ref[...], w2, dn, preferred_element_type=jnp.float32)
        + lax.dot_general(bcol_ref[...], w3, dn, preferred_element_type=jnp.float32)
        + lax.dot_general(gl, w4, dn, preferred_element_type=jnp.float32)
    )
    p_ref[0] = lax.dot_general(m_row, b2, dn,
                               preferred_element_type=jnp.float32) + 0.5 * cvec
    p_ref[1] = lax.dot_general(m_col, b3, dn,
                               preferred_element_type=jnp.float32) + 0.5 * cvec


def _tc_y_body(v_ref, aself_ref, y_ref):
    y_ref[...] = lax.dot_general(v_ref[...], aself_ref[...],
                                 (((1,), (1,)), ((), ())),
                                 preferred_element_type=jnp.float32)


def kernel(index, value, W_self, b_self, W_row, b_row, W_col, b_col,
           W_glob, b_glob, W, b):
    B, E, D = value.shape
    v2 = value.reshape(E, D)

    cpt = -(-(E // SCHUNK) // NS)
    cpt += (-cpt) % 16
    rows_pad = cpt * NS
    cpw = -(-(E // SCHUNK) // (NC * NS))
    cpw += (-cpw) % 8
    rows_pad = max(rows_pad, cpw * NC * NS)

    idx2 = index[0].T.reshape(2, E // 128, 128)
    pad = rows_pad - E // 128
    idxrc = jnp.pad(idx2, ((0, 0), (0, pad), (0, 0)))

    a_self = pl.pallas_call(
        _tc_aself_body,
        out_shape=jax.ShapeDtypeStruct((D, D), jnp.float32),
    )(W_self, W)

    BLK = 1280
    y_call = pl.pallas_call(
        _tc_y_body,
        grid=(E // BLK,),
        in_specs=[
            pl.BlockSpec((BLK, D), lambda i: (i, 0)),
            pl.BlockSpec((D, D), lambda i: (0, 0)),
        ],
        out_specs=pl.BlockSpec((BLK, D), lambda i: (i, 0)),
        out_shape=jax.ShapeDtypeStruct((E, D), jnp.float32),
    )
    y_edge = y_call(v2, a_self)

    s_tab, cnt_tab = _make_sc_segment_sum(E, D)(v2, idxrc)
    npt = N_PAD // NS
    cnt_n = cnt_tab.reshape(2, NS, 1024)[:, :, :npt].reshape(2, N_PAD)

    node_call = pl.pallas_call(
        _tc_node_body,
        out_shape=jax.ShapeDtypeStruct((2, N_PAD, D), jnp.float32),
    )
    p_tab = node_call(
        s_tab, cnt_n[0].reshape(N_PAD, 1), cnt_n[1].reshape(N_PAD, 1),
        W_row, W_col, W_glob, W,
        b_self.reshape(1, D), b_row.reshape(1, D), b_col.reshape(1, D),
        b_glob.reshape(1, D), b.reshape(1, D),
        jnp.full((1, 1), float(E), jnp.float32),
    )

    out = _make_sc_combine(E, D)(p_tab, idxrc, y_edge)
    return (index, out.reshape(B, E, D))
